# baseline (speedup 1.0000x reference)
import functools

import jax
import jax.numpy as jnp
from jax import lax
from jax.experimental import pallas as pl
from jax.experimental.pallas import tpu as pltpu
from jax.experimental.pallas import tpu_sc as plsc

B = 4096
C = 1000
T = 10
P = 5
V = T * P
D = 128
ROW = V * D
ROWW = ROW // 2
VPAD = 64
NW = 32
EPW = B // NW
CH = 8
NCHUNK = EPW // CH
RB = 2
NK = D // 16
NKB = D // 32
EPS2 = 1e-16
HIMASK = -65536


def _rsqrt16(x):
    i = plsc.bitcast(x, jnp.int32)
    i = jnp.int32(0x5F3759DF) - lax.shift_right_arithmetic(i, 1)
    y = plsc.bitcast(i, jnp.float32)
    for _ in range(2):
        y = y * (jnp.float32(1.5) - jnp.float32(0.5) * x * y * y)
    return y


def _widen(acc):
    u = plsc.bitcast(acc, jnp.int32)
    lo = plsc.bitcast(lax.shift_left(u, 16), jnp.float32)
    hi = plsc.bitcast(jnp.bitwise_and(u, jnp.int32(HIMASK)), jnp.float32)
    return lo + hi


def _packpair(x, y):
    half = jnp.int32(0x8000)
    him = jnp.int32(HIMASK)
    xr = jnp.bitwise_and(plsc.bitcast(x, jnp.int32) + half, him)
    yr = jnp.bitwise_and(plsc.bitcast(y, jnp.int32) + half, him)
    return jnp.bitwise_or(lax.shift_right_logical(xr, 16), yr)


def _sc_body(gemd, cid, ptab, out, tabA, tabB, idx0, idx1, b_all, b16,
             rows0, rows1, rowf0, rowf1, prow0, prow1, dbuf, nbuf, tbuf,
             obuf0, obuf1, sem0, sem1, semf0, semf1, semp0, semp1,
             semo0, semo1):
    cc = lax.axis_index("c")
    sid = lax.axis_index("s")
    wid = sid * 2 + cc
    base = wid * EPW
    iota = lax.iota(jnp.int32, 16)
    xor_masks = [iota ^ m for m in (8, 4, 2, 1)]

    rbase = sid * 64
    npair = jnp.where(sid < 15, 16, 10)

    def conv_rows(rowf, prow, blk, semp):
        def one_row(rr, carry):
            for v in range(V):
                avs = [rowf[rr, pl.ds(v * D + k * 16, 16)]
                       for k in range(NK)]
                for k2 in range(NKB):
                    prow[rr, pl.ds(v * 64 + k2 * 16, 16)] = _packpair(
                        avs[2 * k2], avs[2 * k2 + 1])
            return carry

        lax.fori_loop(0, RB, one_row, 0)
        r0 = rbase + blk * RB

        @pl.when(cc == 0)
        def _():
            pltpu.async_copy(prow, tabA.at[pl.ds(r0, RB)], semp)

        @pl.when(cc == 1)
        def _():
            pltpu.async_copy(prow, tabB.at[pl.ds(r0, RB)], semp)

    pltpu.async_copy(ptab.at[pl.ds(rbase, RB)], rowf0, semf0)

    def conv_pair(i, carry):
        b0 = i * 2
        pltpu.async_copy(ptab.at[pl.ds(rbase + (b0 + 1) * RB, RB)],
                         rowf1, semf1)
        pltpu.make_async_copy(ptab.at[pl.ds(rbase, RB)], rowf0, semf0).wait()

        @pl.when(i > 0)
        def _():
            pltpu.make_async_copy(prow0, tabA.at[pl.ds(0, RB)], semp0).wait()

        conv_rows(rowf0, prow0, b0, semp0)

        @pl.when(b0 + 2 < npair * 2)
        def _():
            pltpu.async_copy(ptab.at[pl.ds(rbase + (b0 + 2) * RB, RB)],
                             rowf0, semf0)

        pltpu.make_async_copy(ptab.at[pl.ds(rbase, RB)], rowf1, semf1).wait()

        @pl.when(i > 0)
        def _():
            pltpu.make_async_copy(prow1, tabA.at[pl.ds(0, RB)], semp1).wait()

        conv_rows(rowf1, prow1, b0 + 1, semp1)
        return carry

    lax.fori_loop(0, npair, conv_pair, 0)
    pltpu.make_async_copy(prow0, tabA.at[pl.ds(0, RB)], semp0).wait()
    pltpu.make_async_copy(prow1, tabA.at[pl.ds(0, RB)], semp1).wait()

    pltpu.sync_copy(gemd.at[pl.ds(base, EPW)], b_all)

    def norm_body(r, carry):
        bks = [b_all[r, pl.ds(k * 16, 16)] for k in range(NK)]
        acc = bks[0] * bks[0]
        for k in range(1, NK):
            acc = acc + bks[k] * bks[k]
        for m in xor_masks:
            tbuf[pl.ds(0, 16)] = acc
            acc = acc + plsc.load_gather(tbuf, [m])
        rnb = _rsqrt16(jnp.maximum(acc, jnp.float32(EPS2)))
        for k in range(NKB):
            b16[r, pl.ds(k * 16, 16)] = _packpair(
                bks[2 * k] * rnb, bks[2 * k + 1] * rnb)
        return carry

    lax.fori_loop(0, EPW, norm_body, 0)
    plsc.subcore_barrier()

    def issue_gather(idx, rows, sem):
        @pl.when(cc == 0)
        def _():
            pltpu.async_copy(tabA.at[idx], rows, sem)

        @pl.when(cc == 1)
        def _():
            pltpu.async_copy(tabB.at[idx], rows, sem)

    def compute_chunk(c, rows_v, obuf, semo):
        def elem_body(e, ecarry):
            ce = c * CH + e
            bks = [plsc.bitcast(b16[ce, pl.ds(k * 16, 16)], jnp.bfloat16)
                   for k in range(NKB)]
            for g in range(4):
                nj = 16 if g < 3 else V - 48
                accd = []
                accn = []
                for j in range(nj):
                    av = plsc.bitcast(
                        rows_v[e, pl.ds((g * 16 + j) * 64, 16)],
                        jnp.bfloat16)
                    accd.append(av * bks[0])
                    accn.append(av * av)
                for k in range(1, NKB):
                    for j in range(nj):
                        av = plsc.bitcast(
                            rows_v[e, pl.ds((g * 16 + j) * 64 + k * 16, 16)],
                            jnp.bfloat16)
                        accd[j] = accd[j] + av * bks[k]
                        accn[j] = accn[j] + av * av
                for j in range(nj):
                    dbuf[pl.ds(j * 16, 16)] = _widen(accd[j])
                    nbuf[pl.ds(j * 16, 16)] = _widen(accn[j])
                gidx = iota * 16
                dparts = [plsc.load_gather(dbuf, [gidx + j])
                          for j in range(16)]
                nparts = [plsc.load_gather(nbuf, [gidx + j])
                          for j in range(16)]
                while len(dparts) > 1:
                    dparts = [dparts[i] + dparts[i + 1]
                              for i in range(0, len(dparts), 2)]
                    nparts = [nparts[i] + nparts[i + 1]
                              for i in range(0, len(nparts), 2)]
                rna = _rsqrt16(jnp.maximum(nparts[0], jnp.float32(EPS2)))
                obuf[e, pl.ds(g * 16, 16)] = dparts[0] * rna
            return ecarry

        lax.fori_loop(0, CH, elem_body, 0)
        pltpu.async_copy(obuf, out.at[pl.ds(base + c * CH, CH)], semo)

    pltpu.sync_copy(cid.at[pl.ds(base, CH)], idx0)
    issue_gather(idx0, rows0, sem0)

    def pair_body(i, carry):
        c = i * 2
        pltpu.sync_copy(cid.at[pl.ds(base + (c + 1) * CH, CH)], idx1)
        issue_gather(idx1, rows1, sem1)
        pltpu.make_async_copy(tabA.at[idx0], rows0, sem0).wait()

        @pl.when(i > 0)
        def _():
            pltpu.make_async_copy(obuf0, out.at[pl.ds(base, CH)], semo0).wait()

        compute_chunk(c, rows0, obuf0, semo0)

        @pl.when(c + 2 < NCHUNK)
        def _():
            pltpu.sync_copy(cid.at[pl.ds(base + (c + 2) * CH, CH)], idx0)
            issue_gather(idx0, rows0, sem0)

        pltpu.make_async_copy(tabA.at[idx1], rows1, sem1).wait()

        @pl.when(i > 0)
        def _():
            pltpu.make_async_copy(obuf1, out.at[pl.ds(base, CH)], semo1).wait()

        compute_chunk(c + 1, rows1, obuf1, semo1)
        return carry

    lax.fori_loop(0, NCHUNK // 2, pair_body, 0)
    pltpu.make_async_copy(obuf0, out.at[pl.ds(base, CH)], semo0).wait()
    pltpu.make_async_copy(obuf1, out.at[pl.ds(base, CH)], semo1).wait()


@jax.jit
def _cosine(gemd, cid, ptab):
    mesh = plsc.VectorSubcoreMesh(core_axis_name="c", subcore_axis_name="s")
    run = functools.partial(
        pl.kernel,
        mesh=mesh,
        out_type=[
            jax.ShapeDtypeStruct((B, VPAD), jnp.float32),
            jax.ShapeDtypeStruct((C, ROWW), jnp.int32),
            jax.ShapeDtypeStruct((C, ROWW), jnp.int32),
        ],
        compiler_params=pltpu.CompilerParams(needs_layout_passes=False),
        scratch_types=[
            pltpu.VMEM((CH,), jnp.int32),
            pltpu.VMEM((CH,), jnp.int32),
            pltpu.VMEM((EPW, D), jnp.float32),
            pltpu.VMEM((EPW, D // 2), jnp.int32),
            pltpu.VMEM((CH, ROWW), jnp.int32),
            pltpu.VMEM((CH, ROWW), jnp.int32),
            pltpu.VMEM((RB, ROW), jnp.float32),
            pltpu.VMEM((RB, ROW), jnp.float32),
            pltpu.VMEM((RB, ROWW), jnp.int32),
            pltpu.VMEM((RB, ROWW), jnp.int32),
            pltpu.VMEM((256,), jnp.float32),
            pltpu.VMEM((256,), jnp.float32),
            pltpu.VMEM((16,), jnp.float32),
            pltpu.VMEM((CH, VPAD), jnp.float32),
            pltpu.VMEM((CH, VPAD), jnp.float32),
            pltpu.SemaphoreType.DMA,
            pltpu.SemaphoreType.DMA,
            pltpu.SemaphoreType.DMA,
            pltpu.SemaphoreType.DMA,
            pltpu.SemaphoreType.DMA,
            pltpu.SemaphoreType.DMA,
            pltpu.SemaphoreType.DMA,
            pltpu.SemaphoreType.DMA,
        ],
    )(_sc_body)
    return run(gemd, cid, ptab)[0]


def kernel(graph_emd, cluster_id, prompts):
    cid = cluster_id.astype(jnp.int32)
    out = _cosine(graph_emd, cid, prompts.reshape(C, ROW))
    return out[:, :V].reshape(B, T, P)


# speedup vs baseline: 1.0941x; 1.0941x over previous
import functools

import jax
import jax.numpy as jnp
from jax import lax
from jax.experimental import pallas as pl
from jax.experimental.pallas import tpu as pltpu
from jax.experimental.pallas import tpu_sc as plsc

B = 4096
C = 1000
T = 10
P = 5
V = T * P
D = 128
ROWW = V * D // 2
VPAD = 64
NW = 32
EPW = B // NW
CH = 8
NCHUNK = EPW // CH
NK = D // 16
NKB = D // 32
EPS2 = 1e-16
HIMASK = -65536


def _rsqrt16(x):
    i = plsc.bitcast(x, jnp.int32)
    i = jnp.int32(0x5F3759DF) - lax.shift_right_arithmetic(i, 1)
    y = plsc.bitcast(i, jnp.float32)
    for _ in range(3):
        y = y * (jnp.float32(1.5) - jnp.float32(0.5) * x * y * y)
    return y


def _widen(acc):
    u = plsc.bitcast(acc, jnp.int32)
    lo = plsc.bitcast(lax.shift_left(u, 16), jnp.float32)
    hi = plsc.bitcast(jnp.bitwise_and(u, jnp.int32(HIMASK)), jnp.float32)
    return lo + hi


def _sc_body(gemd, cid, ptab, out, idx0, idx1, b_all, rows0, rows1, dbuf,
             nbuf, tbuf, obuf0, obuf1, sem0, sem1, semo0, semo1):
    wid = lax.axis_index("s") * 2 + lax.axis_index("c")
    base = wid * EPW
    iota = lax.iota(jnp.int32, 16)
    xor_masks = [iota ^ m for m in (8, 4, 2, 1)]

    pltpu.sync_copy(gemd.at[pl.ds(base, EPW)], b_all)

    def norm_body(r, carry):
        bks = [b_all[r, pl.ds(k * 16, 16)] for k in range(NK)]
        acc = bks[0] * bks[0]
        for k in range(1, NK):
            acc = acc + bks[k] * bks[k]
        for m in xor_masks:
            tbuf[pl.ds(0, 16)] = acc
            acc = acc + plsc.load_gather(tbuf, [m])
        rnb = _rsqrt16(jnp.maximum(acc, jnp.float32(EPS2)))
        for k in range(NK):
            b_all[r, pl.ds(k * 16, 16)] = bks[k] * rnb
        return carry

    lax.fori_loop(0, EPW, norm_body, 0)

    def compute_chunk(c, rows_v, obuf, semo):
        def elem_body(e, ecarry):
            ce = c * CH + e
            bks = [b_all[ce, pl.ds(k * 16, 16)] for k in range(NK)]
            for g in range(4):
                nj = 16 if g < 3 else V - 48
                accd = []
                accn = []
                for j in range(nj):
                    av = rows_v[e, pl.ds((g * 16 + j) * D, 16)]
                    accd.append(av * bks[0])
                    accn.append(av * av)
                for k in range(1, NK):
                    for j in range(nj):
                        av = rows_v[e, pl.ds((g * 16 + j) * D + k * 16, 16)]
                        accd[j] = accd[j] + av * bks[k]
                        accn[j] = accn[j] + av * av
                for j in range(nj):
                    dbuf[pl.ds(j * 16, 16)] = accd[j]
                    nbuf[pl.ds(j * 16, 16)] = accn[j]
                gidx = iota * 16
                dparts = [plsc.load_gather(dbuf, [gidx + j])
                          for j in range(16)]
                nparts = [plsc.load_gather(nbuf, [gidx + j])
                          for j in range(16)]
                while len(dparts) > 1:
                    dparts = [dparts[i] + dparts[i + 1]
                              for i in range(0, len(dparts), 2)]
                    nparts = [nparts[i] + nparts[i + 1]
                              for i in range(0, len(nparts), 2)]
                rna = _rsqrt16(jnp.maximum(nparts[0], jnp.float32(EPS2)))
                obuf[e, pl.ds(g * 16, 16)] = dparts[0] * rna
            return ecarry

        lax.fori_loop(0, CH, elem_body, 0)
        pltpu.async_copy(obuf, out.at[pl.ds(base + c * CH, CH)], semo)

    pltpu.sync_copy(cid.at[pl.ds(base, CH)], idx0)
    pltpu.async_copy(ptab.at[idx0], rows0, sem0)

    def pair_body(i, carry):
        c = i * 2
        pltpu.sync_copy(cid.at[pl.ds(base + (c + 1) * CH, CH)], idx1)
        pltpu.async_copy(ptab.at[idx1], rows1, sem1)
        pltpu.make_async_copy(ptab.at[idx0], rows0, sem0).wait()

        @pl.when(i > 0)
        def _():
            pltpu.make_async_copy(obuf0, out.at[pl.ds(base, CH)], semo0).wait()

        compute_chunk(c, rows0, obuf0, semo0)

        @pl.when(c + 2 < NCHUNK)
        def _():
            pltpu.sync_copy(cid.at[pl.ds(base + (c + 2) * CH, CH)], idx0)
            pltpu.async_copy(ptab.at[idx0], rows0, sem0)

        pltpu.make_async_copy(ptab.at[idx1], rows1, sem1).wait()

        @pl.when(i > 0)
        def _():
            pltpu.make_async_copy(obuf1, out.at[pl.ds(base, CH)], semo1).wait()

        compute_chunk(c + 1, rows1, obuf1, semo1)
        return carry

    lax.fori_loop(0, NCHUNK // 2, pair_body, 0)
    pltpu.make_async_copy(obuf0, out.at[pl.ds(base, CH)], semo0).wait()
    pltpu.make_async_copy(obuf1, out.at[pl.ds(base, CH)], semo1).wait()


@jax.jit
def _cosine(gemd, cid, ptab):
    mesh = plsc.VectorSubcoreMesh(core_axis_name="c", subcore_axis_name="s")
    run = functools.partial(
        pl.kernel,
        mesh=mesh,
        out_type=jax.ShapeDtypeStruct((B, VPAD), jnp.float32),
        compiler_params=pltpu.CompilerParams(needs_layout_passes=False),
        scratch_types=[
            pltpu.VMEM((CH,), jnp.int32),
            pltpu.VMEM((CH,), jnp.int32),
            pltpu.VMEM((EPW, D), jnp.float32),
            pltpu.VMEM((CH, V * D), jnp.float32),
            pltpu.VMEM((CH, V * D), jnp.float32),
            pltpu.VMEM((256,), jnp.float32),
            pltpu.VMEM((256,), jnp.float32),
            pltpu.VMEM((16,), jnp.float32),
            pltpu.VMEM((CH, VPAD), jnp.float32),
            pltpu.VMEM((CH, VPAD), jnp.float32),
            pltpu.SemaphoreType.DMA,
            pltpu.SemaphoreType.DMA,
            pltpu.SemaphoreType.DMA,
            pltpu.SemaphoreType.DMA,
        ],
    )(_sc_body)
    return run(gemd, cid, ptab)


def kernel(graph_emd, cluster_id, prompts):
    cid = cluster_id.astype(jnp.int32)
    out = _cosine(graph_emd, cid, prompts.reshape(C, V * D))
    return out[:, :V].reshape(B, T, P)
